# baseline (speedup 1.0000x reference)
import jax
import jax.numpy as jnp
from jax.experimental import pallas as pl
from jax.experimental.pallas import tpu as pltpu

_EMBED = 256
_HIDDEN = 512
_KEY = 64
_VALUE = 64
_MEM = 1000000
_OUT = 256
_B = 16
_TOPK = 3
_THRESH = 0.5

_BLK = 8000
_NBLK = _MEM // _BLK
_NEG = float("-inf")
_BIGI = 2**30


def _stream_kernel(se_ref, w1_ref, b1_ref, w2_ref, b2_ref, ones_ref, keys_ref,
                   scores_ref, idx_ref, qn_ref):
    i = pl.program_id(0)

    @pl.when(i == 0)
    def _init():
        h = jnp.maximum(
            jnp.dot(se_ref[...], w1_ref[...], preferred_element_type=jnp.float32)
            + b1_ref[...], 0.0)
        q = jnp.dot(h, w2_ref[...], preferred_element_type=jnp.float32) + b2_ref[...]
        qnorm = jnp.sqrt(jnp.sum(q * q, axis=1, keepdims=True))
        qn_ref[...] = q / (qnorm + 1e-8)
        scores_ref[...] = jnp.full((_B, 8), _NEG, jnp.float32)
        idx_ref[...] = jnp.zeros((_B, 8), jnp.int32)

    keys = keys_ref[...]
    sims = jax.lax.dot_general(
        qn_ref[...], keys, (((1,), (1,)), ((), ())),
        preferred_element_type=jnp.float32)
    keys2 = keys * keys
    norms2 = jax.lax.dot_general(
        ones_ref[...], keys2, (((1,), (1,)), ((), ())),
        preferred_element_type=jnp.float32)
    inv = 1.0 / (jnp.sqrt(norms2[0:1, :]) + 1e-8)
    sims = sims * inv

    prev_s = scores_ref[...]
    prev_i = idx_ref[...]
    blockmax = jnp.max(sims, axis=1)
    need = jnp.any(blockmax > prev_s[:, 2])

    @pl.when(need)
    def _merge():
        col = jax.lax.broadcasted_iota(jnp.int32, (_B, _BLK), 1)
        s_work = sims
        new_s, new_i = [], []
        for _ in range(_TOPK):
            m = jnp.max(s_work, axis=1, keepdims=True)
            am = jnp.min(jnp.where(s_work == m, col, _BIGI), axis=1,
                         keepdims=True)
            new_s.append(m)
            new_i.append(am + i * _BLK)
            s_work = jnp.where(col == am, _NEG, s_work)
        cand_s = jnp.concatenate([prev_s[:, :_TOPK]] + new_s, axis=1)
        cand_i = jnp.concatenate([prev_i[:, :_TOPK]] + new_i, axis=1)
        out_s, out_i = [], []
        for _ in range(_TOPK):
            m2 = jnp.max(cand_s, axis=1, keepdims=True)
            pi = jnp.min(jnp.where(cand_s == m2, cand_i, _BIGI), axis=1,
                         keepdims=True)
            out_s.append(m2)
            out_i.append(pi)
            cand_s = jnp.where(cand_i == pi, _NEG, cand_s)
        scores_ref[...] = jnp.concatenate(
            out_s + [jnp.full((_B, 8 - _TOPK), _NEG, jnp.float32)], axis=1)
        idx_ref[...] = jnp.concatenate(
            out_i + [jnp.zeros((_B, 8 - _TOPK), jnp.int32)], axis=1)


def _finish_kernel(idx_ref, scores_ref, se_ref, ow_ref, ob_ref, vals_hbm,
                   out_ref, gather_ref, sem):
    for b in range(_B):
        for j in range(_TOPK):
            r = idx_ref[b, j]
            pltpu.make_async_copy(
                vals_hbm.at[pl.ds(r, 1), :],
                gather_ref.at[pl.ds(j * _B + b, 1), :],
                sem.at[j * _B + b]).start()
    for b in range(_B):
        for j in range(_TOPK):
            pltpu.make_async_copy(
                vals_hbm.at[pl.ds(0, 1), :],
                gather_ref.at[pl.ds(j * _B + b, 1), :],
                sem.at[j * _B + b]).wait()
    scores3 = scores_ref[:, :_TOPK]
    w = jnp.where(scores3 >= _THRESH, scores3, 0.0)
    readout = (w[:, 0:1] * gather_ref[0 * _B:1 * _B, :]
               + w[:, 1:2] * gather_ref[1 * _B:2 * _B, :]
               + w[:, 2:3] * gather_ref[2 * _B:3 * _B, :])
    out = (jnp.dot(se_ref[...], ow_ref[:_EMBED, :],
                   preferred_element_type=jnp.float32)
           + jnp.dot(readout, ow_ref[_EMBED:, :],
                     preferred_element_type=jnp.float32)
           + ob_ref[...])
    out_ref[...] = out


def kernel(state_embedding, k_w1, k_b1, k_w2, k_b2, mem_keys, mem_values,
           out_w, out_b):
    b1 = k_b1.reshape(1, _HIDDEN)
    b2 = k_b2.reshape(1, _KEY)
    ob = out_b.reshape(1, _OUT)
    ones_row = jnp.ones((8, _KEY), jnp.float32)

    full = lambda shape: pl.BlockSpec(shape, lambda i: (0, 0))
    scores, idx = pl.pallas_call(
        _stream_kernel,
        grid=(_NBLK,),
        in_specs=[
            full((_B, _EMBED)),
            full((_EMBED, _HIDDEN)),
            full((1, _HIDDEN)),
            full((_HIDDEN, _KEY)),
            full((1, _KEY)),
            full((8, _KEY)),
            pl.BlockSpec((_BLK, _KEY), lambda i: (i, 0)),
        ],
        out_specs=[full((_B, 8)), full((_B, 8))],
        out_shape=[
            jax.ShapeDtypeStruct((_B, 8), jnp.float32),
            jax.ShapeDtypeStruct((_B, 8), jnp.int32),
        ],
        scratch_shapes=[pltpu.VMEM((_B, _KEY), jnp.float32)],
        compiler_params=pltpu.CompilerParams(
            dimension_semantics=("arbitrary",)),
    )(state_embedding, k_w1, b1, k_w2, b2, ones_row, mem_keys)

    out = pl.pallas_call(
        _finish_kernel,
        in_specs=[
            pl.BlockSpec(memory_space=pltpu.SMEM),
            pl.BlockSpec(memory_space=pltpu.VMEM),
            pl.BlockSpec(memory_space=pltpu.VMEM),
            pl.BlockSpec(memory_space=pltpu.VMEM),
            pl.BlockSpec(memory_space=pltpu.VMEM),
            pl.BlockSpec(memory_space=pl.ANY),
        ],
        out_specs=pl.BlockSpec(memory_space=pltpu.VMEM),
        out_shape=jax.ShapeDtypeStruct((_B, _OUT), jnp.float32),
        scratch_shapes=[
            pltpu.VMEM((_TOPK * _B, _VALUE), jnp.float32),
            pltpu.SemaphoreType.DMA((_TOPK * _B,)),
        ],
    )(idx, scores, state_embedding, out_w, ob, mem_values)
    return out
